# baseline (speedup 1.0000x reference)
import jax
import jax.numpy as jnp
from jax import lax
from jax.experimental import pallas as pl
from jax.experimental.pallas import tpu as pltpu
from jax.experimental.pallas import tpu_sc as plsc

D_MODEL = 2048
SEQ_LEN = 2048
LANES = 16

_NC = 2
_NS = 16
_NW = _NC * _NS
SC_ROWS = 1536
TC_ROWS = SEQ_LEN - SC_ROWS
_ROWS_PER_W = SC_ROWS // _NW
_CHUNK = 16
_NCHUNK = _ROWS_PER_W // _CHUNK

_mesh = plsc.VectorSubcoreMesh(
    core_axis_name="c", subcore_axis_name="s",
    num_cores=_NC, num_subcores=_NS)


def _emb_add_body(x_hbm, table_hbm, out_hbm,
                  xi_v, xf_v, buf0, buf1, gsem0, gsem1, osem0, osem1):
    wid = lax.axis_index("s") * _NC + lax.axis_index("c")
    base = wid * _ROWS_PER_W

    pltpu.sync_copy(x_hbm, xi_v)
    bufs = (buf0, buf1)
    gsems = (gsem0, gsem1)
    osems = (osem0, osem1)
    pltpu.async_copy(table_hbm.at[xi_v[pl.ds(base, _CHUNK)]], bufs[0], gsems[0])

    @pl.loop(0, SEQ_LEN // LANES)
    def _cvt(j):
        xf_v[pl.ds(j * LANES, LANES)] = (
            xi_v[pl.ds(j * LANES, LANES)].astype(jnp.float32))

    for g in range(_NCHUNK):
        b = g & 1
        pltpu.make_async_copy(
            table_hbm.at[xi_v[pl.ds(base + g * _CHUNK, _CHUNK)]],
            bufs[b], gsems[b]).wait()
        if g + 1 < _NCHUNK:
            nb = (g + 1) & 1
            if g >= 1:
                pltpu.make_async_copy(
                    bufs[nb],
                    out_hbm.at[pl.ds(base + (g - 1) * _CHUNK, _CHUNK)],
                    osems[nb],
                ).wait()
            pltpu.async_copy(
                table_hbm.at[xi_v[pl.ds(base + (g + 1) * _CHUNK, _CHUNK)]],
                bufs[nb], gsems[nb])

        @pl.loop(0, D_MODEL // LANES)
        def _add_row_vec(j, _b=b):
            xv = xf_v[pl.ds(j * LANES, LANES)]
            for r in range(_CHUNK):
                plsc.addupdate(bufs[_b].at[r, pl.ds(j * LANES, LANES)], xv)

        pltpu.async_copy(
            bufs[b], out_hbm.at[pl.ds(base + g * _CHUNK, _CHUNK)], osems[b])

    if _NCHUNK > 1:
        pltpu.make_async_copy(
            bufs[(_NCHUNK - 2) & 1],
            out_hbm.at[pl.ds(base + (_NCHUNK - 2) * _CHUNK, _CHUNK)],
            osems[(_NCHUNK - 2) & 1]).wait()
    pltpu.make_async_copy(
        bufs[(_NCHUNK - 1) & 1],
        out_hbm.at[pl.ds(base + (_NCHUNK - 1) * _CHUNK, _CHUNK)],
        osems[(_NCHUNK - 1) & 1]).wait()


_SCRATCH = [
    pltpu.VMEM((SEQ_LEN,), jnp.int32),
    pltpu.VMEM((D_MODEL,), jnp.float32),
    pltpu.VMEM((_CHUNK, D_MODEL), jnp.float32),
    pltpu.VMEM((_CHUNK, D_MODEL), jnp.float32),
    pltpu.SemaphoreType.DMA,
    pltpu.SemaphoreType.DMA,
    pltpu.SemaphoreType.DMA,
    pltpu.SemaphoreType.DMA,
]

_emb_add_sc = pl.kernel(
    _emb_add_body,
    out_type=jax.ShapeDtypeStruct((SEQ_LEN, D_MODEL), jnp.float32),
    mesh=_mesh,
    scratch_types=_SCRATCH,
)


def _tc_body(idx_ref, table_row, xf_row, out_row):
    out_row[...] = table_row[...] + xf_row[...]


_tc_gather = pl.pallas_call(
    _tc_body,
    grid_spec=pltpu.PrefetchScalarGridSpec(
        num_scalar_prefetch=1,
        grid=(TC_ROWS,),
        in_specs=[
            pl.BlockSpec((1, 1, D_MODEL),
                         lambda i, idx: (idx[SC_ROWS + i], 0, 0)),
            pl.BlockSpec((1, 1, D_MODEL), lambda i, idx: (0, 0, 0)),
        ],
        out_specs=pl.BlockSpec((1, 1, D_MODEL), lambda i, idx: (i, 0, 0)),
    ),
    out_shape=jax.ShapeDtypeStruct((TC_ROWS, 1, D_MODEL), jnp.float32),
)


def kernel(x, embedding_weight):
    sc_out = _emb_add_sc(x, embedding_weight)
    xf = x.astype(jnp.float32).reshape(1, 1, D_MODEL)
    tc_out = _tc_gather(
        x, embedding_weight.reshape(8192, 1, D_MODEL), xf)
    return lax.dynamic_update_slice(
        sc_out, tc_out.reshape(TC_ROWS, D_MODEL), (SC_ROWS, 0))


# speedup vs baseline: 2.7182x; 2.7182x over previous
import jax
import jax.numpy as jnp
from jax import lax
from jax.experimental import pallas as pl
from jax.experimental.pallas import tpu as pltpu
from jax.experimental.pallas import tpu_sc as plsc

D_MODEL = 2048
SEQ_LEN = 2048
LANES = 16

_NC = 2
_NS = 16
_NW = _NC * _NS
SC_ROWS = 1536
TC_ROWS = SEQ_LEN - SC_ROWS
_ROWS_PER_W = SC_ROWS // _NW
_CHUNK = 16
_NCHUNK = _ROWS_PER_W // _CHUNK

_mesh = plsc.VectorSubcoreMesh(
    core_axis_name="c", subcore_axis_name="s",
    num_cores=_NC, num_subcores=_NS)


def _emb_add_body(x_hbm, table_hbm, out_hbm,
                  xi_v, xf_v, buf0, buf1, gsem0, gsem1, osem0, osem1):
    wid = lax.axis_index("s") * _NC + lax.axis_index("c")
    base = wid * _ROWS_PER_W

    pltpu.sync_copy(x_hbm, xi_v)
    bufs = (buf0, buf1)
    gsems = (gsem0, gsem1)
    osems = (osem0, osem1)
    pltpu.async_copy(table_hbm.at[xi_v[pl.ds(base, _CHUNK)]], bufs[0], gsems[0])

    @pl.loop(0, SEQ_LEN // LANES)
    def _cvt(j):
        xf_v[pl.ds(j * LANES, LANES)] = (
            xi_v[pl.ds(j * LANES, LANES)].astype(jnp.float32))

    for g in range(_NCHUNK):
        b = g & 1
        pltpu.make_async_copy(
            table_hbm.at[xi_v[pl.ds(base + g * _CHUNK, _CHUNK)]],
            bufs[b], gsems[b]).wait()
        if g + 1 < _NCHUNK:
            nb = (g + 1) & 1
            if g >= 1:
                pltpu.make_async_copy(
                    bufs[nb],
                    out_hbm.at[pl.ds(base + (g - 1) * _CHUNK, _CHUNK)],
                    osems[nb],
                ).wait()
            pltpu.async_copy(
                table_hbm.at[xi_v[pl.ds(base + (g + 1) * _CHUNK, _CHUNK)]],
                bufs[nb], gsems[nb])

        @pl.loop(0, D_MODEL // LANES)
        def _add_row_vec(j, _b=b):
            xv = xf_v[pl.ds(j * LANES, LANES)]
            for r in range(_CHUNK):
                plsc.addupdate(bufs[_b].at[r, pl.ds(j * LANES, LANES)], xv)

        pltpu.async_copy(
            bufs[b], out_hbm.at[pl.ds(base + g * _CHUNK, _CHUNK)], osems[b])

    if _NCHUNK > 1:
        pltpu.make_async_copy(
            bufs[(_NCHUNK - 2) & 1],
            out_hbm.at[pl.ds(base + (_NCHUNK - 2) * _CHUNK, _CHUNK)],
            osems[(_NCHUNK - 2) & 1]).wait()
    pltpu.make_async_copy(
        bufs[(_NCHUNK - 1) & 1],
        out_hbm.at[pl.ds(base + (_NCHUNK - 1) * _CHUNK, _CHUNK)],
        osems[(_NCHUNK - 1) & 1]).wait()


_SCRATCH = [
    pltpu.VMEM((SEQ_LEN,), jnp.int32),
    pltpu.VMEM((D_MODEL,), jnp.float32),
    pltpu.VMEM((_CHUNK, D_MODEL), jnp.float32),
    pltpu.VMEM((_CHUNK, D_MODEL), jnp.float32),
    pltpu.SemaphoreType.DMA,
    pltpu.SemaphoreType.DMA,
    pltpu.SemaphoreType.DMA,
    pltpu.SemaphoreType.DMA,
]

_emb_add_sc = pl.kernel(
    _emb_add_body,
    out_type=jax.ShapeDtypeStruct((SEQ_LEN, D_MODEL), jnp.float32),
    mesh=_mesh,
    scratch_types=_SCRATCH,
)


_K = 8


def _tc_body(idx_ref, *refs):
    xf_row = refs[_K]
    out_blk = refs[_K + 1]
    for k in range(_K):
        out_blk[0, k, :] = refs[k][0, 0, :] + xf_row[0, 0, :]


def _tc_row_spec(k):
    return pl.BlockSpec(
        (1, 1, D_MODEL), lambda i, idx, _k=k: (idx[SC_ROWS + i * _K + _k], 0, 0))


_tc_gather = pl.pallas_call(
    _tc_body,
    grid_spec=pltpu.PrefetchScalarGridSpec(
        num_scalar_prefetch=1,
        grid=(TC_ROWS // _K,),
        in_specs=(
            [_tc_row_spec(k) for k in range(_K)]
            + [pl.BlockSpec((1, 1, D_MODEL), lambda i, idx: (0, 0, 0))]
        ),
        out_specs=pl.BlockSpec((1, _K, D_MODEL), lambda i, idx: (i, 0, 0)),
    ),
    out_shape=jax.ShapeDtypeStruct((TC_ROWS // _K, _K, D_MODEL), jnp.float32),
)


def kernel(x, embedding_weight):
    sc_out = _emb_add_sc(x, embedding_weight)
    xf = x.astype(jnp.float32).reshape(1, 1, D_MODEL)
    table3 = embedding_weight.reshape(8192, 1, D_MODEL)
    tc_out = _tc_gather(x, *([table3] * _K), xf)
    return lax.dynamic_update_slice(
        sc_out, tc_out.reshape(TC_ROWS, D_MODEL), (SC_ROWS, 0))


# speedup vs baseline: 8.6464x; 3.1809x over previous
import jax
import jax.numpy as jnp
from jax import lax
from jax.experimental import pallas as pl
from jax.experimental.pallas import tpu as pltpu
from jax.experimental.pallas import tpu_sc as plsc

D_MODEL = 2048
SEQ_LEN = 2048
LANES = 16

_NC = 2
_NS = 16
_NW = _NC * _NS
_ROWS_PER_W = SEQ_LEN // _NW
_CHUNK = 16
_NCHUNK = _ROWS_PER_W // _CHUNK
_NBUF = 3

_mesh = plsc.VectorSubcoreMesh(
    core_axis_name="c", subcore_axis_name="s",
    num_cores=_NC, num_subcores=_NS)


def _emb_add_body(x_hbm, table_hbm, out_hbm,
                  xi_v, xf_v, buf0, buf1, buf2,
                  gsem0, gsem1, gsem2, osem0, osem1, osem2):
    wid = lax.axis_index("s") * _NC + lax.axis_index("c")
    base = wid * _ROWS_PER_W

    bufs = (buf0, buf1, buf2)
    gsems = (gsem0, gsem1, gsem2)
    osems = (osem0, osem1, osem2)

    def idx_vec(g):
        return xi_v[pl.ds(base + g * _CHUNK, _CHUNK)]

    def out_rows(g):
        return out_hbm.at[pl.ds(base + g * _CHUNK, _CHUNK)]

    pltpu.sync_copy(x_hbm, xi_v)
    pltpu.async_copy(table_hbm.at[idx_vec(0)], bufs[0], gsems[0])
    pltpu.async_copy(table_hbm.at[idx_vec(1)], bufs[1], gsems[1])

    @pl.loop(0, SEQ_LEN // LANES)
    def _cvt(j):
        xf_v[pl.ds(j * LANES, LANES)] = (
            xi_v[pl.ds(j * LANES, LANES)].astype(jnp.float32))

    for g in range(_NCHUNK):
        b = g % _NBUF
        nxt = g + 2
        if nxt < _NCHUNK:
            nb = nxt % _NBUF
            if nxt >= _NBUF:
                pltpu.make_async_copy(
                    bufs[nb], out_rows(nxt - _NBUF), osems[nb]).wait()
            pltpu.async_copy(table_hbm.at[idx_vec(nxt)], bufs[nb], gsems[nb])

        pltpu.make_async_copy(table_hbm.at[idx_vec(g)], bufs[b], gsems[b]).wait()

        @pl.loop(0, D_MODEL // LANES)
        def _add_row_vec(j, _b=b):
            xv = xf_v[pl.ds(j * LANES, LANES)]
            for r in range(_CHUNK):
                plsc.addupdate(bufs[_b].at[r, pl.ds(j * LANES, LANES)], xv)

        pltpu.async_copy(bufs[b], out_rows(g), osems[b])

    for g in range(max(0, _NCHUNK - _NBUF), _NCHUNK):
        b = g % _NBUF
        pltpu.make_async_copy(bufs[b], out_rows(g), osems[b]).wait()


_SCRATCH = [
    pltpu.VMEM((SEQ_LEN,), jnp.int32),
    pltpu.VMEM((D_MODEL,), jnp.float32),
    pltpu.VMEM((_CHUNK, D_MODEL), jnp.float32),
    pltpu.VMEM((_CHUNK, D_MODEL), jnp.float32),
    pltpu.VMEM((_CHUNK, D_MODEL), jnp.float32),
    pltpu.SemaphoreType.DMA,
    pltpu.SemaphoreType.DMA,
    pltpu.SemaphoreType.DMA,
    pltpu.SemaphoreType.DMA,
    pltpu.SemaphoreType.DMA,
    pltpu.SemaphoreType.DMA,
]

_emb_add = pl.kernel(
    _emb_add_body,
    out_type=jax.ShapeDtypeStruct((SEQ_LEN, D_MODEL), jnp.float32),
    mesh=_mesh,
    scratch_types=_SCRATCH,
)


def kernel(x, embedding_weight):
    return _emb_add(x, embedding_weight)


# speedup vs baseline: 9.1178x; 1.0545x over previous
import jax
import jax.numpy as jnp
from jax import lax
from jax.experimental import pallas as pl
from jax.experimental.pallas import tpu as pltpu
from jax.experimental.pallas import tpu_sc as plsc

D_MODEL = 2048
SEQ_LEN = 2048
LANES = 16

_NC = 2
_NS = 16
_NW = _NC * _NS
_ROWS_PER_W = SEQ_LEN // _NW
_CHUNK = 16
_NCHUNK = _ROWS_PER_W // _CHUNK

_mesh = plsc.VectorSubcoreMesh(
    core_axis_name="c", subcore_axis_name="s",
    num_cores=_NC, num_subcores=_NS)


def _emb_add_body(x_hbm, table_hbm, out_hbm,
                  xi_v, xf_v, buf0, buf1, gsem0, gsem1, osem0, osem1):
    wid = lax.axis_index("s") * _NC + lax.axis_index("c")
    base = wid * _ROWS_PER_W

    pltpu.sync_copy(x_hbm, xi_v)
    bufs = (buf0, buf1)
    gsems = (gsem0, gsem1)
    osems = (osem0, osem1)
    pltpu.async_copy(table_hbm.at[xi_v[pl.ds(base, _CHUNK)]], bufs[0], gsems[0])

    @plsc.parallel_loop(0, SEQ_LEN // LANES)
    def _cvt(j):
        xf_v[pl.ds(j * LANES, LANES)] = (
            xi_v[pl.ds(j * LANES, LANES)].astype(jnp.float32))

    for g in range(_NCHUNK):
        b = g & 1
        pltpu.make_async_copy(
            table_hbm.at[xi_v[pl.ds(base + g * _CHUNK, _CHUNK)]],
            bufs[b], gsems[b]).wait()
        if g + 1 < _NCHUNK:
            nb = (g + 1) & 1
            if g >= 1:
                pltpu.make_async_copy(
                    bufs[nb],
                    out_hbm.at[pl.ds(base + (g - 1) * _CHUNK, _CHUNK)],
                    osems[nb],
                ).wait()
            pltpu.async_copy(
                table_hbm.at[xi_v[pl.ds(base + (g + 1) * _CHUNK, _CHUNK)]],
                bufs[nb], gsems[nb])

        for h in range(2):
            @plsc.parallel_loop(0, D_MODEL // LANES)
            def _add_row_vec(j, _b=b, _h=h):
                xv = xf_v[pl.ds(j * LANES, LANES)]
                for r in range(_h * (_CHUNK // 2), (_h + 1) * (_CHUNK // 2)):
                    plsc.addupdate(bufs[_b].at[r, pl.ds(j * LANES, LANES)], xv)

            pltpu.async_copy(
                bufs[b].at[pl.ds(h * (_CHUNK // 2), _CHUNK // 2)],
                out_hbm.at[pl.ds(base + g * _CHUNK + h * (_CHUNK // 2),
                                 _CHUNK // 2)],
                osems[b])

    pltpu.make_async_copy(
        bufs[0], out_hbm.at[pl.ds(base + (_NCHUNK - 2) * _CHUNK, _CHUNK)],
        osems[0]).wait()
    pltpu.make_async_copy(
        bufs[1], out_hbm.at[pl.ds(base + (_NCHUNK - 1) * _CHUNK, _CHUNK)],
        osems[1]).wait()


_SCRATCH = [
    pltpu.VMEM((SEQ_LEN,), jnp.int32),
    pltpu.VMEM((D_MODEL,), jnp.float32),
    pltpu.VMEM((_CHUNK, D_MODEL), jnp.float32),
    pltpu.VMEM((_CHUNK, D_MODEL), jnp.float32),
    pltpu.SemaphoreType.DMA,
    pltpu.SemaphoreType.DMA,
    pltpu.SemaphoreType.DMA,
    pltpu.SemaphoreType.DMA,
]

_emb_add = pl.kernel(
    _emb_add_body,
    out_type=jax.ShapeDtypeStruct((SEQ_LEN, D_MODEL), jnp.float32),
    mesh=_mesh,
    scratch_types=_SCRATCH,
)


def kernel(x, embedding_weight):
    return _emb_add(x, embedding_weight)
